# Initial kernel scaffold; baseline (speedup 1.0000x reference)
import jax
import jax.numpy as jnp
from jax.experimental import pallas as pl


def kernel(input_bytes, params):
    raise NotImplementedError("write your pallas kernel here")


# speedup vs baseline: 1.7492x; 1.7492x over previous
import functools
import math

import jax
import jax.numpy as jnp
from jax.experimental import pallas as pl
from jax.experimental.pallas import tpu as pltpu

H = 128
NHEAD = 4
HD = H // NHEAD
FF = 512
NLAYERS = 2
VOCAB = 256
QB = 512


def _dot_t(a, w):
    return jax.lax.dot_general(
        a, w, (((1,), (1,)), ((), ())), preferred_element_type=jnp.float32)


def _dot(a, b):
    return jax.lax.dot_general(
        a, b, (((1,), (0,)), ((), ())), preferred_element_type=jnp.float32)


def _ln(x, g, b, eps=1e-5):
    m = jnp.mean(x, axis=-1, keepdims=True)
    c = x - m
    v = jnp.mean(c * c, axis=-1, keepdims=True)
    return c * jax.lax.rsqrt(v + eps) * g + b


def _attention(h, wqkv, bqkv, wo, bo, seq):
    qkv = _dot_t(h, wqkv) + bqkv
    scale = 1.0 / math.sqrt(HD)
    row_chunks = []
    for qi in range(0, seq, QB):
        head_outs = []
        q_rows = qkv[qi:qi + QB, :]
        for hh in range(NHEAD):
            qh = q_rows[:, hh * HD:(hh + 1) * HD]
            kh = qkv[:, H + hh * HD:H + (hh + 1) * HD]
            vh = qkv[:, 2 * H + hh * HD:2 * H + (hh + 1) * HD]
            s = jax.lax.dot_general(
                qh, kh, (((1,), (1,)), ((), ())),
                preferred_element_type=jnp.float32) * scale
            m = jnp.max(s, axis=-1, keepdims=True)
            e = jnp.exp(s - m)
            p = e / jnp.sum(e, axis=-1, keepdims=True)
            head_outs.append(_dot(p, vh))
        row_chunks.append(jnp.concatenate(head_outs, axis=1))
    o = jnp.concatenate(row_chunks, axis=0)
    return _dot_t(o, wo) + bo


def _fwd_kernel(bytes_ref, emb_ref, pos_ref, lng_ref, lnb_ref,
                l0_wqkv, l0_bqkv, l0_wo, l0_bo, l0_ln1g, l0_ln1b,
                l0_w1, l0_b1, l0_w2, l0_b2, l0_ln2g, l0_ln2b,
                l1_wqkv, l1_bqkv, l1_wo, l1_bo, l1_ln1g, l1_ln1b,
                l1_w1, l1_b1, l1_w2, l1_b2, l1_ln2g, l1_ln2b,
                wout_ref, bout_ref, out_ref):
    seq = bytes_ref.shape[1]
    bcol = bytes_ref[0]
    onehot = (bcol == jax.lax.broadcasted_iota(
        jnp.int32, (seq, VOCAB), 1)).astype(jnp.float32)
    h = _dot(onehot, emb_ref[...]) + pos_ref[...]
    h = _ln(h, lng_ref[...], lnb_ref[...])

    layer_refs = [
        (l0_wqkv, l0_bqkv, l0_wo, l0_bo, l0_ln1g, l0_ln1b,
         l0_w1, l0_b1, l0_w2, l0_b2, l0_ln2g, l0_ln2b),
        (l1_wqkv, l1_bqkv, l1_wo, l1_bo, l1_ln1g, l1_ln1b,
         l1_w1, l1_b1, l1_w2, l1_b2, l1_ln2g, l1_ln2b),
    ]
    for (wqkv, bqkv, wo, bo, ln1g, ln1b,
         w1, b1, w2, b2, ln2g, ln2b) in layer_refs:
        att = _attention(h, wqkv[...], bqkv[...], wo[...], bo[...], seq)
        h = _ln(h + att, ln1g[...], ln1b[...])
        ff = _dot_t(jnp.maximum(_dot_t(h, w1[...]) + b1[...], 0.0), w2[...]) + b2[...]
        h = _ln(h + ff, ln2g[...], ln2b[...])

    out_ref[0] = _dot_t(h, wout_ref[...]) + bout_ref[...]


@jax.jit
def _run(bytes3d, flat_weights):
    b, seq, _ = bytes3d.shape
    full = lambda shp: pl.BlockSpec(shp, lambda i: (0,) * len(shp))
    in_specs = [pl.BlockSpec((1, seq, 1), lambda i: (i, 0, 0))]
    in_specs += [full(w.shape) for w in flat_weights]
    return pl.pallas_call(
        _fwd_kernel,
        grid=(b,),
        in_specs=in_specs,
        out_specs=pl.BlockSpec((1, seq, VOCAB), lambda i: (i, 0, 0)),
        out_shape=jax.ShapeDtypeStruct((b, seq, VOCAB), jnp.float32),
        compiler_params=pltpu.CompilerParams(
            dimension_semantics=("parallel",),
            vmem_limit_bytes=110 * 1024 * 1024,
        ),
    )(bytes3d, *flat_weights)


def kernel(input_bytes, params):
    b, seq = input_bytes.shape
    row = lambda x: x.reshape(1, -1)
    flat = [params['emb'], params['pos_emb'][:seq],
            row(params['ln_g']), row(params['ln_b'])]
    for lp in params['layers']:
        flat += [lp['Wqkv'], row(lp['bqkv']), lp['Wo'], row(lp['bo']),
                 row(lp['ln1_g']), row(lp['ln1_b']),
                 lp['W1'], row(lp['b1']), lp['W2'], row(lp['b2']),
                 row(lp['ln2_g']), row(lp['ln2_b'])]
    flat += [params['Wout'], row(params['bout'])]
    bytes3d = input_bytes.reshape(b, seq, 1).astype(jnp.int32)
    return _run(bytes3d, flat)
